# baseline (speedup 1.0000x reference)
import jax
import jax.numpy as jnp
from jax import lax
from jax.experimental import pallas as pl
from jax.experimental.pallas import tpu as pltpu
from jax.experimental.pallas import tpu_sc as plsc

DIM = 64
BATCH = 16384
N_FIELDS = 26
B_TOTAL = BATCH * N_FIELDS

_info = plsc.get_sparse_core_info()
_NC, _NS = _info.num_cores, _info.num_subcores
NW = _NC * _NS
B_PER_W = B_TOTAL // NW
CHUNK = 128
GROUP = 1024
N_FIRE = GROUP // CHUNK
N_GROUP = B_PER_W // GROUP


def _emb_body(weight_hbm, idx_hbm, out_hbm, idx_v, rows_v, sem):
    wid = lax.axis_index("s") * _NC + lax.axis_index("c")
    base = wid * B_PER_W
    pltpu.sync_copy(idx_hbm.at[pl.ds(base, B_PER_W)], idx_v)

    def group(g, carry):
        goff = g * GROUP
        copies = []
        for j in range(N_FIRE):
            copies.append(pltpu.async_copy(
                weight_hbm.at[idx_v.at[pl.ds(goff + j * CHUNK, CHUNK)]],
                rows_v.at[pl.ds(j * CHUNK, CHUNK)],
                sem))
        for c in copies:
            c.wait()
        pltpu.sync_copy(rows_v, out_hbm.at[pl.ds(base + goff, GROUP)])
        return carry

    lax.fori_loop(0, N_GROUP, group, 0)


@jax.jit
def kernel(idx, weight):
    idx_flat = idx.reshape(-1).astype(jnp.int32)
    out = pl.kernel(
        _emb_body,
        out_type=jax.ShapeDtypeStruct((B_TOTAL, DIM), jnp.float32),
        mesh=plsc.VectorSubcoreMesh(core_axis_name="c", subcore_axis_name="s"),
        scratch_types=[
            pltpu.VMEM((B_PER_W,), jnp.int32),
            pltpu.VMEM((GROUP, DIM), jnp.float32),
            pltpu.SemaphoreType.DMA,
        ],
        compiler_params=pltpu.CompilerParams(use_tc_tiling_on_sc=False),
    )(weight, idx_flat)
    return out.reshape(BATCH, N_FIELDS, DIM)


# speedup vs baseline: 1.0002x; 1.0002x over previous
import jax
import jax.numpy as jnp
from jax import lax
from jax.experimental import pallas as pl
from jax.experimental.pallas import tpu as pltpu
from jax.experimental.pallas import tpu_sc as plsc

DIM = 64
BATCH = 16384
N_FIELDS = 26
B_TOTAL = BATCH * N_FIELDS

_info = plsc.get_sparse_core_info()
_NC, _NS = _info.num_cores, _info.num_subcores
NW = _NC * _NS
B_PER_W = B_TOTAL // NW
CHUNK = 128
GROUP = 512
N_FIRE = GROUP // CHUNK
N_GROUP = B_PER_W // GROUP


def _emb_body(weight_hbm, idx_hbm, out_hbm,
              idx_v, rows0, rows1, sg0, sg1, sw0, sw1):
    wid = lax.axis_index("s") * _NC + lax.axis_index("c")
    base = wid * B_PER_W
    pltpu.sync_copy(idx_hbm.at[pl.ds(base, B_PER_W)], idx_v)

    def fire(g, buf, sem):
        for j in range(N_FIRE):
            pltpu.async_copy(
                weight_hbm.at[idx_v.at[pl.ds(g * GROUP + j * CHUNK, CHUNK)]],
                buf.at[pl.ds(j * CHUNK, CHUNK)],
                sem)

    def drain_gather(buf, sem):
        for j in range(N_FIRE):
            pltpu.make_async_copy(
                weight_hbm.at[pl.ds(0, CHUNK)],
                buf.at[pl.ds(j * CHUNK, CHUNK)],
                sem).wait()

    def wb_start(g, buf, sem):
        pltpu.async_copy(buf, out_hbm.at[pl.ds(base + g * GROUP, GROUP)], sem)

    def wb_wait(g, buf, sem):
        pltpu.make_async_copy(
            buf, out_hbm.at[pl.ds(base + g * GROUP, GROUP)], sem).wait()

    fire(0, rows0, sg0)
    fire(1, rows1, sg1)

    def pair(i, carry):
        g = 2 * i
        drain_gather(rows0, sg0)
        wb_start(g, rows0, sw0)
        drain_gather(rows1, sg1)
        wb_start(g + 1, rows1, sw1)
        wb_wait(g, rows0, sw0)
        fire(g + 2, rows0, sg0)
        wb_wait(g + 1, rows1, sw1)
        fire(g + 3, rows1, sg1)
        return carry

    lax.fori_loop(0, N_GROUP // 2 - 1, pair, 0)

    g_last = N_GROUP - 2
    drain_gather(rows0, sg0)
    wb_start(g_last, rows0, sw0)
    drain_gather(rows1, sg1)
    wb_start(g_last + 1, rows1, sw1)
    wb_wait(g_last, rows0, sw0)
    wb_wait(g_last + 1, rows1, sw1)


@jax.jit
def kernel(idx, weight):
    idx_flat = idx.reshape(-1).astype(jnp.int32)
    out = pl.kernel(
        _emb_body,
        out_type=jax.ShapeDtypeStruct((B_TOTAL, DIM), jnp.float32),
        mesh=plsc.VectorSubcoreMesh(core_axis_name="c", subcore_axis_name="s"),
        scratch_types=[
            pltpu.VMEM((B_PER_W,), jnp.int32),
            pltpu.VMEM((GROUP, DIM), jnp.float32),
            pltpu.VMEM((GROUP, DIM), jnp.float32),
            pltpu.SemaphoreType.DMA,
            pltpu.SemaphoreType.DMA,
            pltpu.SemaphoreType.DMA,
            pltpu.SemaphoreType.DMA,
        ],
        compiler_params=pltpu.CompilerParams(use_tc_tiling_on_sc=False),
    )(weight, idx_flat)
    return out.reshape(BATCH, N_FIELDS, DIM)
